# baseline (speedup 1.0000x reference)
import functools

import jax
import jax.numpy as jnp
from jax import lax
from jax.experimental import pallas as pl
from jax.experimental.pallas import tpu as pltpu
from jax.experimental.pallas import tpu_sc as plsc

BATCH = 4096
HIST = 200
EMB = 64

NC = 2
NS = 16
NW = NC * NS

BW = BATCH // NW
L = 16


def _make_gather():
    mesh = plsc.VectorSubcoreMesh(
        core_axis_name="c", subcore_axis_name="s", num_cores=NC, num_subcores=NS
    )

    @functools.partial(
        pl.kernel,
        mesh=mesh,
        compiler_params=pltpu.CompilerParams(use_tc_tiling_on_sc=False, needs_layout_passes=False),
        out_type=jax.ShapeDtypeStruct((HIST, EMB, BATCH), jnp.float32),
        scratch_types=[
            pltpu.VMEM((HIST, BW), jnp.int32),
            pltpu.VMEM((2, BW, EMB), jnp.float32),
            pltpu.VMEM((2, EMB, BW), jnp.float32),
            pltpu.SemaphoreType.DMA,
            pltpu.SemaphoreType.DMA,
            pltpu.SemaphoreType.DMA,
            pltpu.SemaphoreType.DMA,
        ],
    )
    def gather_kernel(xt_hbm, table_hbm, out_hbm, idx_v, g_v, t_v,
                      gsem0, gsem1, osem0, osem1):
        wid = lax.axis_index("s") * NC + lax.axis_index("c")
        wb = wid * BW
        pltpu.sync_copy(xt_hbm.at[:, pl.ds(wb, BW)], idx_v)

        gsems = (gsem0, gsem1)
        osems = (osem0, osem1)

        def start_gather(h, b):
            pltpu.async_copy(table_hbm.at[idx_v.at[h]], g_v.at[b], gsems[b])

        def drain_gather(b):
            pltpu.make_async_copy(
                table_hbm.at[idx_v.at[0]], g_v.at[b], gsems[b]
            ).wait()

        def start_out(h, b):
            pltpu.async_copy(
                t_v.at[b], out_hbm.at[h, :, pl.ds(wb, BW)], osems[b]
            )

        def wait_out(b):
            pltpu.make_async_copy(
                t_v.at[b], out_hbm.at[0, :, pl.ds(wb, BW)], osems[b]
            ).wait()

        def transpose(b):
            def erow(e, _):
                for tg in range(BW // L):
                    rows = lax.iota(jnp.int32, L) + (tg * L)
                    cols = jnp.full((L,), 0, jnp.int32) + e
                    vals = plsc.load_gather(g_v.at[b], [rows, cols])
                    t_v[b, e, pl.ds(tg * L, L)] = vals
                return 0
            lax.fori_loop(0, EMB, erow, 0)

        start_gather(0, 0)

        def outer(ho, _):
            for b in range(2):
                h = ho * 2 + b
                drain_gather(b)
                @pl.when(h + 1 < HIST)
                def _():
                    start_gather(h + 1, 1 - b)
                @pl.when(h >= 2)
                def _():
                    wait_out(b)
                transpose(b)
                start_out(h, b)
            return 0

        lax.fori_loop(0, HIST // 2, outer, 0)
        wait_out(0)
        wait_out(1)

    return gather_kernel


_gather = _make_gather()


def kernel(x, table):
    out_t = _gather(x.astype(jnp.int32).T, table)
    return out_t.transpose(2, 0, 1)


# speedup vs baseline: 1.5278x; 1.5278x over previous
import functools

import jax
import jax.numpy as jnp
from jax import lax
from jax.experimental import pallas as pl
from jax.experimental.pallas import tpu as pltpu
from jax.experimental.pallas import tpu_sc as plsc

BATCH = 4096
HIST = 200
EMB = 64

NC = 2
NS = 16
NW = NC * NS

BW = BATCH // NW


def _make_gather():
    mesh = plsc.VectorSubcoreMesh(
        core_axis_name="c", subcore_axis_name="s", num_cores=NC, num_subcores=NS
    )

    @functools.partial(
        pl.kernel,
        mesh=mesh,
        compiler_params=pltpu.CompilerParams(
            use_tc_tiling_on_sc=False, needs_layout_passes=False
        ),
        out_type=jax.ShapeDtypeStruct((BATCH, HIST, EMB), jnp.float32),
        scratch_types=[
            pltpu.VMEM((HIST, BW), jnp.int32),
            pltpu.VMEM((2, BW, EMB), jnp.float32),
            pltpu.SemaphoreType.DMA,
            pltpu.SemaphoreType.DMA,
            pltpu.SemaphoreType.DMA,
            pltpu.SemaphoreType.DMA,
        ],
    )
    def gather_kernel(xt_hbm, table_hbm, out_hbm, idx_v, g_v,
                      gsem0, gsem1, osem0, osem1):
        wid = lax.axis_index("s") * NC + lax.axis_index("c")
        wb = wid * BW
        pltpu.sync_copy(xt_hbm.at[:, pl.ds(wb, BW)], idx_v)

        gsems = (gsem0, gsem1)
        osems = (osem0, osem1)

        def start_gather(h, b):
            pltpu.async_copy(table_hbm.at[idx_v.at[h]], g_v.at[b], gsems[b])

        def drain_gather(b):
            pltpu.make_async_copy(
                table_hbm.at[idx_v.at[0]], g_v.at[b], gsems[b]
            ).wait()

        def start_out(h, b):
            pltpu.async_copy(
                g_v.at[b], out_hbm.at[pl.ds(wb, BW), h], osems[b]
            )

        def wait_out(b):
            pltpu.make_async_copy(
                g_v.at[b], out_hbm.at[pl.ds(wb, BW), 0], osems[b]
            ).wait()

        start_gather(0, 0)

        def outer(ho, _):
            for b in range(2):
                h = ho * 2 + b
                drain_gather(b)
                @pl.when(h >= 2)
                def _():
                    wait_out(b)
                @pl.when(h + 1 < HIST)
                def _():
                    start_gather(h + 1, 1 - b)
                start_out(h, b)
            return 0

        lax.fori_loop(0, HIST // 2, outer, 0)
        wait_out(0)
        wait_out(1)

    return gather_kernel


_gather = _make_gather()


def kernel(x, table):
    return _gather(x.astype(jnp.int32).T, table)
